# baseline (speedup 1.0000x reference)
import functools

import jax
import jax.numpy as jnp
from jax import lax
from jax.experimental import pallas as pl
from jax.experimental.pallas import tpu as pltpu
from jax.experimental.pallas import tpu_sc as plsc

N_NODES = 10000
D = 128
N_EDGES = 320000

NC = 2
NS = 16
NW = NC * NS
CHUNK = 128
CHUNKS_PER_TILE = 80
EDGES_PER_TILE = CHUNK * CHUNKS_PER_TILE
E_PAD = EDGES_PER_TILE * NW
N_PAD = 10240
ROWS_PER_TILE = N_PAD // NS
N_SUPER = 2
SB_CHUNKS = CHUNKS_PER_TILE // N_SUPER


def _make_sc_messages():
    mesh = plsc.VectorSubcoreMesh(core_axis_name="c", subcore_axis_name="s")

    @functools.partial(
        pl.kernel,
        mesh=mesh,
        out_type=jax.ShapeDtypeStruct((NC * N_PAD, D), jnp.float32),
        scratch_types=[
            pltpu.VMEM((SB_CHUNKS, CHUNK), jnp.int32),
            pltpu.VMEM((SB_CHUNKS, CHUNK), jnp.int32),
            pltpu.VMEM((SB_CHUNKS, CHUNK), jnp.float32),
            pltpu.VMEM((CHUNK, D), jnp.float32),
            pltpu.VMEM((CHUNK, D), jnp.float32),
            pltpu.VMEM_SHARED((N_PAD, D), jnp.float32),
            pltpu.SemaphoreType.DMA,
            pltpu.SemaphoreType.DMA,
        ],
    )
    def body(x_hbm, src_hbm, dst_hbm, e_hbm, out_hbm,
             idx_s, idx_d, e_all, sa, da, acc, sem_a, sem_b):
        cid = lax.axis_index("c")
        sid = lax.axis_index("s")
        wid = cid * NS + sid

        zero = jnp.zeros((16,), jnp.float32)

        def zrow(i, carry):
            for r in range(D // 16):
                sa[i, pl.ds(r * 16, 16)] = zero
            return carry

        lax.fori_loop(0, CHUNK, zrow, 0)
        r0 = sid * ROWS_PER_TILE
        for t in range(ROWS_PER_TILE // CHUNK):
            pltpu.sync_copy(sa, acc.at[pl.ds(r0 + t * CHUNK, CHUNK)])
        plsc.subcore_barrier()

        def gather_pair(c, bs, bd, sem):
            pltpu.async_copy(x_hbm.at[idx_s.at[c]], bs, sem)
            pltpu.async_copy(x_hbm.at[idx_d.at[c]], bd, sem)

        def wait_pair(bs, bd, sem):
            pltpu.make_async_copy(x_hbm.at[idx_s.at[0]], bs, sem).wait()
            pltpu.make_async_copy(x_hbm.at[idx_d.at[0]], bd, sem).wait()

        def scale_scatter(c, bs, bd):
            def scale(g, inner):
                ev16 = e_all[c, pl.ds(g * 16, 16)]
                i0 = g * 16
                for j in range(16):
                    eb = jnp.full((16,), ev16[j], jnp.float32)
                    for r in range(D // 16):
                        sl = pl.ds(r * 16, 16)
                        bs[i0 + j, sl] = bs[i0 + j, sl] * eb
                        bd[i0 + j, sl] = bd[i0 + j, sl] * eb
                return inner

            lax.fori_loop(0, CHUNK // 16, scale, 0)
            pltpu.sync_copy(bs, acc.at[idx_d.at[c]], add=True)
            pltpu.sync_copy(bd, acc.at[idx_s.at[c]], add=True)

        def chunk(c, carry):
            wait_pair(sa, da, sem_a)
            scale_scatter(c, sa, da)
            c2 = jnp.minimum(c + 1, SB_CHUNKS - 1)
            gather_pair(c2, sa, da, sem_a)
            return carry

        for s in range(N_SUPER):
            row0 = wid * CHUNKS_PER_TILE + s * SB_CHUNKS
            pltpu.sync_copy(src_hbm.at[pl.ds(row0, SB_CHUNKS)], idx_s)
            pltpu.sync_copy(dst_hbm.at[pl.ds(row0, SB_CHUNKS)], idx_d)
            pltpu.sync_copy(e_hbm.at[pl.ds(row0, SB_CHUNKS)], e_all)
            gather_pair(0, sa, da, sem_a)
            lax.fori_loop(0, SB_CHUNKS, chunk, 0)
            wait_pair(sa, da, sem_a)

        plsc.subcore_barrier()
        out_base = cid * N_PAD + r0
        pltpu.sync_copy(acc.at[pl.ds(r0, ROWS_PER_TILE)],
                        out_hbm.at[pl.ds(out_base, ROWS_PER_TILE)])

    return body


_SC_CACHE = []


def _sc_messages():
    if not _SC_CACHE:
        _SC_CACHE.append(_make_sc_messages())
    return _SC_CACHE[0]


_R = 1000


def _mlp_body(mi_ref, x_ref, w1a_ref, w1b_ref, vecs_ref, w2_ref, out_ref):
    mi = mi_ref[0] + mi_ref[1]
    h = (
        jnp.dot(mi, w1a_ref[...], preferred_element_type=jnp.float32,
                precision=lax.Precision.HIGHEST)
        + jnp.dot(x_ref[...], w1b_ref[...], preferred_element_type=jnp.float32,
                  precision=lax.Precision.HIGHEST)
        + vecs_ref[0:1, :]
    )
    mean = jnp.mean(h, axis=1, keepdims=True)
    var = jnp.mean((h - mean) ** 2, axis=1, keepdims=True)
    h = (h - mean) * lax.rsqrt(var + 1e-5) * vecs_ref[1:2, :] + vecs_ref[2:3, :]
    h = jnp.tanh(h)
    out_ref[...] = (
        jnp.dot(h, w2_ref[...], preferred_element_type=jnp.float32,
                precision=lax.Precision.HIGHEST)
        + vecs_ref[3:4, :]
    )


def _mlp(mi2, x, w1a, w1b, vecs, w2):
    grid = (N_NODES // _R,)
    return pl.pallas_call(
        _mlp_body,
        grid=grid,
        in_specs=[
            pl.BlockSpec((2, _R, D), lambda i: (0, i, 0)),
            pl.BlockSpec((_R, D), lambda i: (i, 0)),
            pl.BlockSpec((D, D), lambda i: (0, 0)),
            pl.BlockSpec((D, D), lambda i: (0, 0)),
            pl.BlockSpec((8, D), lambda i: (0, 0)),
            pl.BlockSpec((D, D), lambda i: (0, 0)),
        ],
        out_specs=pl.BlockSpec((_R, D), lambda i: (i, 0)),
        out_shape=jax.ShapeDtypeStruct((N_NODES, D), jnp.float32),
    )(mi2, x, w1a, w1b, vecs, w2)


def kernel(x, e, edge_index, W1, b1, g1, beta1, W2, b2):
    src = edge_index[0].astype(jnp.int32)
    dst = edge_index[1].astype(jnp.int32)
    pad = E_PAD - N_EDGES
    src = jnp.pad(src, (0, pad)).reshape(NW * CHUNKS_PER_TILE, CHUNK)
    dst = jnp.pad(dst, (0, pad)).reshape(NW * CHUNKS_PER_TILE, CHUNK)
    ep = jnp.pad(e, (0, pad)).reshape(NW * CHUNKS_PER_TILE, CHUNK)
    partials = _sc_messages()(x, src, dst, ep)
    mi2 = partials.reshape(2, N_PAD, D)
    vecs = (
        jnp.zeros((8, D), jnp.float32)
        .at[0].set(b1).at[1].set(g1).at[2].set(beta1).at[3].set(b2)
    )
    return _mlp(mi2, x, W1[:D], W1[D:], vecs, W2)
